# baseline (speedup 1.0000x reference)
import jax
import jax.numpy as jnp
from jax.experimental import pallas as pl
from jax.experimental.pallas import tpu as pltpu

_EMBED = 64
_HEADS = 4
_MODEL_CAT_RATE = 0.02
_ID_CAT_RATE = 0.36
_BM = 512
_F32 = jnp.float32


def _dot(a, b):
    return jnp.dot(a, b, preferred_element_type=_F32)


def _row_normalize(z):
    n = jnp.sqrt(jnp.sum(z * z, axis=1, keepdims=True))
    return z / jnp.maximum(n, 1e-12)


_MK1_BM = 256


def _mk1_body(imf_ref, wi_ref, bi_ref, tf_ref, wt_ref, bt_ref,
              g_img_ui_ref, g_txt_ui_ref, g_img_iu_ref, g_txt_iu_ref,
              item_emb_ref, user_emb_ref,
              o_big, o_uid, o_iid):
    i = pl.program_id(0)
    bm = o_big.shape[0]
    imgf = _dot(imf_ref[...], wi_ref[...]) + bi_ref[...]
    txtf = _dot(tf_ref[...], wt_ref[...]) + bt_ref[...]
    iuid = _dot(g_img_ui_ref[...], item_emb_ref[...])
    tuid = _dot(g_txt_ui_ref[...], item_emb_ref[...])
    iiid = _dot(g_img_iu_ref[...], user_emb_ref[...])
    tiid = _dot(g_txt_iu_ref[...], user_emb_ref[...])
    o_uid[...] = jnp.concatenate([iuid, tuid], axis=1)
    o_iid[...] = jnp.concatenate([iiid, tiid], axis=1)
    sl = pl.ds(i * bm, bm)
    o_big[...] = jnp.concatenate(
        [imgf, txtf, 0.5 * (iiid + tiid), 0.5 * (iuid + tuid),
         item_emb_ref[sl, :], user_emb_ref[sl, :]],
        axis=1).astype(jnp.bfloat16)


def _mk1(image_feats, w_image_trans, b_image_trans, text_feats, w_text_trans,
         b_text_trans, g_img_ui, g_txt_ui, g_img_iu, g_txt_iu,
         item_id_emb, user_id_emb, bm=_MK1_BM):
    m = g_img_ui.shape[0]
    kf = image_feats.shape[1]
    kt = text_feats.shape[1]
    c = _EMBED
    row = lambda k: pl.BlockSpec((bm, k), lambda i: (i, 0))
    const = lambda shape: pl.BlockSpec(shape, lambda i: (0, 0))
    blk = lambda w: pl.BlockSpec((bm, w), lambda i: (i, 0))
    sds = lambda w, dt=_F32: jax.ShapeDtypeStruct((m, w), dt)
    return pl.pallas_call(
        _mk1_body,
        grid=(m // bm,),
        in_specs=[row(kf), const((kf, c)), const((1, c)),
                  row(kt), const((kt, c)), const((1, c)),
                  row(m), row(m), row(m), row(m),
                  const((m, c)), const((m, c))],
        out_specs=[blk(6 * c), blk(2 * c), blk(2 * c)],
        out_shape=[sds(6 * c, jnp.bfloat16), sds(2 * c), sds(2 * c)],
        compiler_params=pltpu.CompilerParams(
            dimension_semantics=("arbitrary",)),
    )(image_feats, w_image_trans, b_image_trans.reshape(1, c),
      text_feats, w_text_trans, b_text_trans.reshape(1, c),
      g_img_ui, g_txt_ui, g_img_iu, g_txt_iu, item_id_emb, user_id_emb)


def _mk2_body(ui_ref, iu_ref, big_ref, wsum_ref,
              o_uf, o_if, o_ug, o_ig,
              g0_s, uf_s, if_s, g1_s, ug2_s):
    i = pl.program_id(0)
    bmu = ui_ref.shape[0]
    bmi = iu_ref.shape[0]
    m = ui_ref.shape[1]
    c = _EMBED
    nbu = m // bmu
    nbi = m // bmi
    b1, b2, b3 = nbu, nbu + nbi, 2 * nbu + nbi

    bf = jnp.bfloat16

    @pl.when(i == 0)
    def _():
        idm = big_ref[:, 2 * c:4 * c].astype(_F32)
        zi = _row_normalize(_dot(idm[:, :c], wsum_ref[...]))
        zu = _row_normalize(_dot(idm[:, c:], wsum_ref[...]))
        g0_s[...] = (big_ref[:, 4 * c:].astype(_F32)
                     + _ID_CAT_RATE * jnp.concatenate([zi, zu], axis=1))

    @pl.when(i < b1)
    def _():
        sl = pl.ds(i * bmu, bmu)
        g = ui_ref[...].astype(bf)
        uf = _dot(g, big_ref[:, :2 * c])
        o_uf[...] = uf
        uf_s[sl, :] = uf.astype(bf)
        g1_s[sl, :c] = _dot(g, g0_s[:, :c].astype(bf)).astype(bf)

    @pl.when((i >= b1) & (i < b2))
    def _():
        sl = pl.ds((i - b1) * bmi, bmi)
        g = iu_ref[...].astype(bf)
        itf = _dot(g, uf_s[...])
        o_if[...] = itf
        if_s[sl, :] = itf.astype(bf)
        g1_s[sl, c:] = _dot(g, g1_s[:, :c]).astype(bf)

    @pl.when((i >= b2) & (i < b3))
    def _():
        sl = pl.ds((i - b2) * bmu, bmu)
        sm = jax.nn.softmax(_dot(ui_ref[...].astype(bf), g1_s[:, c:]),
                            axis=-1)
        ug2_s[sl, :] = sm.astype(bf)
        o_ug[...] = _final(g0_s[sl, c:], g1_s[sl, :c].astype(_F32), sm,
                           uf_s[sl, :c].astype(_F32),
                           uf_s[sl, c:].astype(_F32))

    @pl.when(i >= b3)
    def _():
        sl = pl.ds((i - b3) * bmi, bmi)
        sm = jax.nn.softmax(_dot(iu_ref[...].astype(bf), ug2_s[...]), axis=-1)
        o_ig[...] = _final(g0_s[sl, :c], g1_s[sl, c:].astype(_F32), sm,
                           if_s[sl, :c].astype(_F32),
                           if_s[sl, c:].astype(_F32))


def _mk2(ui, iu, big, w_sum, bmu=_BM, bmi=_BM):
    m, k = ui.shape
    c = _EMBED
    nbu = m // bmu
    nbi = m // bmi
    b1, b2, b3 = nbu, nbu + nbi, 2 * nbu + nbi
    b_end = b3 + nbi
    const = lambda w: pl.BlockSpec((m, w), lambda i: (0, 0))
    wblk = pl.BlockSpec((c, c), lambda i: (0, 0))

    def ui_map(i):
        return (jnp.where(i < b1, i,
                jnp.where(i < b2, 0,
                jnp.where(i < b3, i - b2, nbu - 1))), 0)

    def iu_map(i):
        return (jnp.where(i < b1, 0,
                jnp.where(i < b2, i - b1,
                jnp.where(i < b3, 0, i - b3))), 0)

    def oseg(s, bm, nb, w=c):
        return pl.BlockSpec((bm, w), lambda i: (jnp.clip(i - s, 0, nb - 1), 0))

    out_sds = jax.ShapeDtypeStruct((m, c), _F32)
    scr = lambda w, dt=_F32: pltpu.VMEM((m, w), dt)
    return pl.pallas_call(
        _mk2_body,
        grid=(b3 + nbi,),
        in_specs=[pl.BlockSpec((bmu, k), ui_map),
                  pl.BlockSpec((bmi, k), iu_map),
                  const(6 * c), wblk],
        out_specs=[oseg(0, bmu, nbu, 2 * c), oseg(b1, bmi, nbi, 2 * c),
                   oseg(b2, bmu, nbu), oseg(b3, bmi, nbi)],
        out_shape=[jax.ShapeDtypeStruct((m, 2 * c), _F32),
                   jax.ShapeDtypeStruct((m, 2 * c), _F32),
                   out_sds, out_sds],
        scratch_shapes=[scr(2 * c), scr(2 * c, jnp.bfloat16),
                        scr(2 * c, jnp.bfloat16), scr(2 * c, jnp.bfloat16),
                        scr(c, jnp.bfloat16)],
        compiler_params=pltpu.CompilerParams(
            dimension_semantics=("arbitrary",)),
    )(ui, iu, big, w_sum)


def _final(g0, g1, g2, fa, fb):
    mean_g = (g0 + g1 + g2) * (1.0 / 3.0)
    return (mean_g + _MODEL_CAT_RATE * _row_normalize(fa)
            + _MODEL_CAT_RATE * _row_normalize(fb))


def kernel(ui_graph, iu_graph, image_ui_graph, image_iu_graph, text_ui_graph,
           text_iu_graph, image_feats, text_feats, w_image_trans, b_image_trans,
           w_text_trans, b_text_trans, user_id_emb, item_id_emb, w_q, w_k, w_cat):
    big, uid2, iid2 = _mk1(image_feats, w_image_trans, b_image_trans,
                           text_feats, w_text_trans, b_text_trans,
                           image_ui_graph, text_ui_graph, image_iu_graph,
                           text_iu_graph, item_id_emb, user_id_emb)
    image_user_id, text_user_id = uid2[:, :_EMBED], uid2[:, _EMBED:]
    image_item_id, text_item_id = iid2[:, :_EMBED], iid2[:, _EMBED:]

    w_sum = w_cat.reshape(_HEADS, _EMBED, _EMBED).sum(0)

    uf2, if2, u_g, i_g = _mk2(ui_graph, iu_graph, big, w_sum)
    image_user_feats, text_user_feats = uf2[:, :_EMBED], uf2[:, _EMBED:]
    image_item_feats, text_item_feats = if2[:, :_EMBED], if2[:, _EMBED:]

    return (u_g, i_g, image_item_feats, text_item_feats, image_user_feats,
            text_user_feats, u_g, i_g, image_user_id, text_user_id,
            image_item_id, text_item_id)


# speedup vs baseline: 1.0517x; 1.0517x over previous
import jax
import jax.numpy as jnp
from jax.experimental import pallas as pl
from jax.experimental.pallas import tpu as pltpu

_EMBED = 64
_HEADS = 4
_MODEL_CAT_RATE = 0.02
_ID_CAT_RATE = 0.36
_BM = 512
_F32 = jnp.float32


def _dot(a, b):
    return jnp.dot(a, b, preferred_element_type=_F32)


def _row_normalize(z):
    n = jnp.sqrt(jnp.sum(z * z, axis=1, keepdims=True))
    return z / jnp.maximum(n, 1e-12)


_MK1_BM = 256


def _mk1_body(imf_ref, wi_ref, bi_ref, tf_ref, wt_ref, bt_ref,
              g_img_ui_ref, g_txt_ui_ref, g_img_iu_ref, g_txt_iu_ref,
              item_emb_ref, user_emb_ref,
              o_ft, o_idm, o_emb2, o_img_uid, o_txt_uid, o_img_iid,
              o_txt_iid):
    i = pl.program_id(0)
    bm = o_ft.shape[0]
    bf = jnp.bfloat16
    imgf = _dot(imf_ref[...].astype(bf), wi_ref[...].astype(bf)) + bi_ref[...]
    txtf = _dot(tf_ref[...].astype(bf), wt_ref[...].astype(bf)) + bt_ref[...]
    o_ft[...] = jnp.concatenate([imgf, txtf], axis=1).astype(bf)
    iemb = item_emb_ref[...].astype(bf)
    uemb = user_emb_ref[...].astype(bf)
    iuid = _dot(g_img_ui_ref[...].astype(bf), iemb)
    tuid = _dot(g_txt_ui_ref[...].astype(bf), iemb)
    iiid = _dot(g_img_iu_ref[...].astype(bf), uemb)
    tiid = _dot(g_txt_iu_ref[...].astype(bf), uemb)
    o_img_uid[...] = iuid
    o_txt_uid[...] = tuid
    o_img_iid[...] = iiid
    o_txt_iid[...] = tiid
    o_idm[...] = jnp.concatenate(
        [0.5 * (iiid + tiid), 0.5 * (iuid + tuid)], 1).astype(jnp.bfloat16)
    sl = pl.ds(i * bm, bm)
    o_emb2[...] = jnp.concatenate([item_emb_ref[sl, :], user_emb_ref[sl, :]],
                                  axis=1).astype(jnp.bfloat16)


def _mk1(image_feats, w_image_trans, b_image_trans, text_feats, w_text_trans,
         b_text_trans, g_img_ui, g_txt_ui, g_img_iu, g_txt_iu,
         item_id_emb, user_id_emb, bm=_MK1_BM):
    m = g_img_ui.shape[0]
    kf = image_feats.shape[1]
    kt = text_feats.shape[1]
    c = _EMBED
    row = lambda k: pl.BlockSpec((bm, k), lambda i: (i, 0))
    const = lambda shape: pl.BlockSpec(shape, lambda i: (0, 0))
    blk = lambda w: pl.BlockSpec((bm, w), lambda i: (i, 0))
    sds = lambda w, dt=_F32: jax.ShapeDtypeStruct((m, w), dt)
    return pl.pallas_call(
        _mk1_body,
        grid=(m // bm,),
        in_specs=[row(kf), const((kf, c)), const((1, c)),
                  row(kt), const((kt, c)), const((1, c)),
                  row(m), row(m), row(m), row(m),
                  const((m, c)), const((m, c))],
        out_specs=[blk(2 * c), blk(2 * c), blk(2 * c),
                   blk(c), blk(c), blk(c), blk(c)],
        out_shape=[sds(2 * c, jnp.bfloat16), sds(2 * c, jnp.bfloat16),
                   sds(2 * c, jnp.bfloat16),
                   sds(c), sds(c), sds(c), sds(c)],
        compiler_params=pltpu.CompilerParams(
            dimension_semantics=("arbitrary",)),
    )(image_feats, w_image_trans, b_image_trans.reshape(1, c),
      text_feats, w_text_trans, b_text_trans.reshape(1, c),
      g_img_ui, g_txt_ui, g_img_iu, g_txt_iu, item_id_emb, user_id_emb)


def _mk2_body(ui_ref, iu_ref, ft_ref, idm_ref, emb2_ref, wsum_ref,
              o_iuf, o_tuf, o_iif, o_tif, o_ug, o_ig,
              g0_s, uf_s, if_s, g1_s, ug2_s):
    i = pl.program_id(0)
    bmu = ui_ref.shape[0]
    bmi = iu_ref.shape[0]
    m = ui_ref.shape[1]
    c = _EMBED
    nbu = m // bmu
    nbi = m // bmi
    b1, b2, b3 = nbu, nbu + nbi, 2 * nbu + nbi

    bf = jnp.bfloat16

    @pl.when(i == 0)
    def _():
        idm = idm_ref[...].astype(_F32)
        zi = _row_normalize(_dot(idm[:, :c], wsum_ref[...]))
        zu = _row_normalize(_dot(idm[:, c:], wsum_ref[...]))
        g0_s[...] = (emb2_ref[...].astype(_F32)
                     + _ID_CAT_RATE * jnp.concatenate([zi, zu], axis=1))

    @pl.when(i < b1)
    def _():
        sl = pl.ds(i * bmu, bmu)
        g = ui_ref[...].astype(bf)
        uf = _dot(g, ft_ref[...])
        o_iuf[...] = uf[:, :c]
        o_tuf[...] = uf[:, c:]
        uf_s[sl, :] = uf.astype(bf)
        g1_s[sl, :c] = _dot(g, g0_s[:, :c].astype(bf)).astype(bf)

    @pl.when((i >= b1) & (i < b2))
    def _():
        sl = pl.ds((i - b1) * bmi, bmi)
        g = iu_ref[...].astype(bf)
        itf = _dot(g, uf_s[...])
        o_iif[...] = itf[:, :c]
        o_tif[...] = itf[:, c:]
        if_s[sl, :] = itf.astype(bf)
        g1_s[sl, c:] = _dot(g, g1_s[:, :c]).astype(bf)

    @pl.when((i >= b2) & (i < b3))
    def _():
        sl = pl.ds((i - b2) * bmu, bmu)
        sm = jax.nn.softmax(_dot(ui_ref[...].astype(bf), g1_s[:, c:]),
                            axis=-1)
        ug2_s[sl, :] = sm.astype(bf)
        o_ug[...] = _final(g0_s[sl, c:], g1_s[sl, :c].astype(_F32), sm,
                           uf_s[sl, :c].astype(_F32),
                           uf_s[sl, c:].astype(_F32))

    @pl.when(i >= b3)
    def _():
        sl = pl.ds((i - b3) * bmi, bmi)
        sm = jax.nn.softmax(_dot(iu_ref[...].astype(bf), ug2_s[...]), axis=-1)
        o_ig[...] = _final(g0_s[sl, :c], g1_s[sl, c:].astype(_F32), sm,
                           if_s[sl, :c].astype(_F32),
                           if_s[sl, c:].astype(_F32))


def _mk2(ui, iu, ft, idm, emb2, w_sum, bmu=_BM, bmi=_BM):
    m, k = ui.shape
    c = _EMBED
    nbu = m // bmu
    nbi = m // bmi
    b1, b2, b3 = nbu, nbu + nbi, 2 * nbu + nbi
    b_end = b3 + nbi
    const = lambda w: pl.BlockSpec((m, w), lambda i: (0, 0))
    wblk = pl.BlockSpec((c, c), lambda i: (0, 0))

    def ui_map(i):
        return (jnp.where(i < b1, i,
                jnp.where(i < b2, 0,
                jnp.where(i < b3, i - b2, nbu - 1))), 0)

    def iu_map(i):
        return (jnp.where(i < b1, 0,
                jnp.where(i < b2, i - b1,
                jnp.where(i < b3, 0, i - b3))), 0)

    def oseg(s, bm, nb):
        return pl.BlockSpec((bm, c), lambda i: (jnp.clip(i - s, 0, nb - 1), 0))

    out_sds = jax.ShapeDtypeStruct((m, c), _F32)
    scr = lambda w, dt=_F32: pltpu.VMEM((m, w), dt)
    return pl.pallas_call(
        _mk2_body,
        grid=(b3 + nbi,),
        in_specs=[pl.BlockSpec((bmu, k), ui_map),
                  pl.BlockSpec((bmi, k), iu_map),
                  const(2 * c), const(2 * c), const(2 * c), wblk],
        out_specs=[oseg(0, bmu, nbu), oseg(0, bmu, nbu),
                   oseg(b1, bmi, nbi), oseg(b1, bmi, nbi),
                   oseg(b2, bmu, nbu), oseg(b3, bmi, nbi)],
        out_shape=[out_sds] * 6,
        scratch_shapes=[scr(2 * c), scr(2 * c, jnp.bfloat16),
                        scr(2 * c, jnp.bfloat16), scr(2 * c, jnp.bfloat16),
                        scr(c, jnp.bfloat16)],
        compiler_params=pltpu.CompilerParams(
            dimension_semantics=("arbitrary",)),
    )(ui, iu, ft, idm, emb2, w_sum)


def _final(g0, g1, g2, fa, fb):
    mean_g = (g0 + g1 + g2) * (1.0 / 3.0)
    return (mean_g + _MODEL_CAT_RATE * _row_normalize(fa)
            + _MODEL_CAT_RATE * _row_normalize(fb))


def kernel(ui_graph, iu_graph, image_ui_graph, image_iu_graph, text_ui_graph,
           text_iu_graph, image_feats, text_feats, w_image_trans, b_image_trans,
           w_text_trans, b_text_trans, user_id_emb, item_id_emb, w_q, w_k, w_cat):
    (ft, idm, emb2, image_user_id, text_user_id, image_item_id,
     text_item_id) = _mk1(image_feats, w_image_trans, b_image_trans,
                          text_feats, w_text_trans, b_text_trans,
                          image_ui_graph, text_ui_graph, image_iu_graph,
                          text_iu_graph, item_id_emb, user_id_emb)

    w_sum = w_cat.reshape(_HEADS, _EMBED, _EMBED).sum(0)

    (image_user_feats, text_user_feats, image_item_feats, text_item_feats,
     u_g, i_g) = _mk2(ui_graph, iu_graph, ft, idm, emb2, w_sum)

    return (u_g, i_g, image_item_feats, text_item_feats, image_user_feats,
            text_user_feats, u_g, i_g, image_user_id, text_user_id,
            image_item_id, text_item_id)
